# baseline (speedup 1.0000x reference)
import functools

import jax
import jax.numpy as jnp
import numpy as np
from jax.experimental import pallas as pl
from jax.experimental.pallas import tpu as pltpu

S = 2048
H = 12
D = 64
RADIUS_SQ = 6.25
BQ = 256
BK = 256
NJ = S // BK
NEG = -1e30
SCALE = float(1.0 / np.sqrt(D))


def _attn_kernel(q_ref, k_ref, v_ref, qspan_ref, kspan_ref, caus_ref,
                 qc_ref, kc_ref, o_ref, bias_ref):
    i = pl.program_id(0)
    h = pl.program_id(1)

    kspan = kspan_ref[...]
    s_q_min = qspan_ref[0, 0]
    s_q_max = qspan_ref[BQ - 1, 0]
    first_same = jnp.sum((kspan < s_q_min).astype(jnp.int32))
    extent = jnp.sum((kspan <= s_q_max).astype(jnp.int32))
    j_hist_end = first_same // BK
    j_end = (extent + BK - 1) // BK

    @pl.when(h == 0)
    def _():
        qspan = qspan_ref[...]
        caus = caus_ref[...]
        qx = qc_ref[:, 0:1]
        qy = qc_ref[:, 1:2]
        qidx = i * BQ + jax.lax.broadcasted_iota(jnp.int32, (BQ, 1), 0)

        def bias_body(j, _):
            ksl = kspan_ref[0:1, pl.ds(j * BK, BK)]
            kx = kc_ref[0:1, pl.ds(j * BK, BK)]
            ky = kc_ref[1:2, pl.ds(j * BK, BK)]
            kidx = j * BK + jax.lax.broadcasted_iota(jnp.int32, (1, BK), 1)
            dist = (qx - kx) ** 2 + (qy - ky) ** 2
            time_ok = (caus == 0) | (qidx >= kidx)
            vis = (ksl < qspan) | ((ksl == qspan) & time_ok
                                   & (dist < RADIUS_SQ))
            bias_ref[:, pl.ds(j * BK, BK)] = jnp.where(vis, 0.0, NEG)
            return 0

        jax.lax.fori_loop(j_hist_end, j_end, bias_body, 0)

    q = q_ref[0] * SCALE

    def hist_body(j, carry):
        m, l, acc = carry
        kb = k_ref[0, pl.ds(j * BK, BK), :]
        s = jax.lax.dot_general(q, kb, (((1,), (1,)), ((), ())),
                                preferred_element_type=jnp.float32)
        m_new = jnp.maximum(m, jnp.max(s, axis=1, keepdims=True))
        p = jnp.exp(s - m_new)
        r = jnp.exp(m - m_new)
        vb = v_ref[0, pl.ds(j * BK, BK), :]
        pv = jax.lax.dot_general(p, vb, (((1,), (0,)), ((), ())),
                                 preferred_element_type=jnp.float32)
        return (m_new, l * r + jnp.sum(p, axis=1, keepdims=True),
                acc * r + pv)

    def same_body(j, carry):
        m, l, acc = carry
        kb = k_ref[0, pl.ds(j * BK, BK), :]
        s = jax.lax.dot_general(q, kb, (((1,), (1,)), ((), ())),
                                preferred_element_type=jnp.float32)
        s = s + bias_ref[:, pl.ds(j * BK, BK)]
        m_new = jnp.maximum(m, jnp.max(s, axis=1, keepdims=True))
        p = jnp.exp(s - m_new)
        r = jnp.exp(m - m_new)
        vb = v_ref[0, pl.ds(j * BK, BK), :]
        pv = jax.lax.dot_general(p, vb, (((1,), (0,)), ((), ())),
                                 preferred_element_type=jnp.float32)
        return (m_new, l * r + jnp.sum(p, axis=1, keepdims=True),
                acc * r + pv)

    m0 = jnp.full((BQ, 1), NEG, dtype=jnp.float32)
    l0 = jnp.zeros((BQ, 1), dtype=jnp.float32)
    a0 = jnp.zeros((BQ, D), dtype=jnp.float32)
    carry = jax.lax.fori_loop(0, j_hist_end, hist_body, (m0, l0, a0))
    m, l, acc = jax.lax.fori_loop(j_hist_end, j_end, same_body, carry)
    o_ref[0] = acc / l


@jax.jit
def kernel(q, k, v, coords, span_ids, is_causal):
    q3 = q[0]
    k3 = k[0]
    v3 = v[0]
    span_col = span_ids.reshape(S, 1)
    span_row = span_ids.reshape(1, S)
    caus_col = is_causal.astype(jnp.int32).reshape(S, 1)
    coords_t = coords.T

    grid = (S // BQ, H)
    out = pl.pallas_call(
        _attn_kernel,
        grid=grid,
        in_specs=[
            pl.BlockSpec((1, BQ, D), lambda i, h: (h, i, 0)),
            pl.BlockSpec((1, S, D), lambda i, h: (h, 0, 0)),
            pl.BlockSpec((1, S, D), lambda i, h: (h, 0, 0)),
            pl.BlockSpec((BQ, 1), lambda i, h: (i, 0)),
            pl.BlockSpec((1, S), lambda i, h: (0, 0)),
            pl.BlockSpec((BQ, 1), lambda i, h: (i, 0)),
            pl.BlockSpec((BQ, 2), lambda i, h: (i, 0)),
            pl.BlockSpec((2, S), lambda i, h: (0, 0)),
        ],
        out_specs=pl.BlockSpec((1, BQ, D), lambda i, h: (h, i, 0)),
        out_shape=jax.ShapeDtypeStruct((H, S, D), jnp.float32),
        scratch_shapes=[pltpu.VMEM((BQ, S), jnp.float32)],
        compiler_params=pltpu.CompilerParams(
            dimension_semantics=("arbitrary", "arbitrary")),
    )(q3, k3, v3, span_col, span_row, caus_col, coords, coords_t)
    return out[None]


# speedup vs baseline: 1.5584x; 1.5584x over previous
import jax
import jax.numpy as jnp
import numpy as np
from jax.experimental import pallas as pl
from jax.experimental.pallas import tpu as pltpu

S = 2048
H = 12
D = 64
RADIUS_SQ = 6.25
BQ = 256
NEG = -1e30
SCALE = float(1.0 / np.sqrt(D))


def _attn_kernel(q_ref, k_ref, v_ref, qspan_ref, kspan_ref, caus_ref,
                 qc_ref, kc_ref, o_ref, bias_ref):
    i = pl.program_id(0)
    h = pl.program_id(1)

    @pl.when(h == 0)
    def _():
        qspan = qspan_ref[...]
        kspan = kspan_ref[...]
        caus = caus_ref[...]
        qx = qc_ref[:, 0:1]
        qy = qc_ref[:, 1:2]
        kx = kc_ref[0:1, :]
        ky = kc_ref[1:2, :]
        qidx = i * BQ + jax.lax.broadcasted_iota(jnp.int32, (BQ, 1), 0)
        kidx = jax.lax.broadcasted_iota(jnp.int32, (1, S), 1)
        dist = (qx - kx) ** 2 + (qy - ky) ** 2
        time_ok = (caus == 0) | (qidx >= kidx)
        vis = (kspan < qspan) | ((kspan == qspan) & time_ok
                                 & (dist < RADIUS_SQ))
        bias_ref[...] = jnp.where(vis, 0.0, NEG)

    q = q_ref[0] * SCALE
    k = k_ref[0]
    v = v_ref[0]
    s = jax.lax.dot_general(q, k, (((1,), (1,)), ((), ())),
                            preferred_element_type=jnp.float32)
    s = s + bias_ref[...]
    m = jnp.max(s, axis=1, keepdims=True)
    p = jnp.exp(s - m)
    l = jnp.sum(p, axis=1, keepdims=True)
    o = jax.lax.dot_general(p, v, (((1,), (0,)), ((), ())),
                            preferred_element_type=jnp.float32)
    o_ref[0] = o / l


@jax.jit
def kernel(q, k, v, coords, span_ids, is_causal):
    q3 = q[0]
    k3 = k[0]
    v3 = v[0]
    span_col = span_ids.reshape(S, 1)
    span_row = span_ids.reshape(1, S)
    caus_col = is_causal.astype(jnp.int32).reshape(S, 1)
    coords_t = coords.T

    grid = (S // BQ, H)
    out = pl.pallas_call(
        _attn_kernel,
        grid=grid,
        in_specs=[
            pl.BlockSpec((1, BQ, D), lambda i, h: (h, i, 0)),
            pl.BlockSpec((1, S, D), lambda i, h: (h, 0, 0)),
            pl.BlockSpec((1, S, D), lambda i, h: (h, 0, 0)),
            pl.BlockSpec((BQ, 1), lambda i, h: (i, 0)),
            pl.BlockSpec((1, S), lambda i, h: (0, 0)),
            pl.BlockSpec((BQ, 1), lambda i, h: (i, 0)),
            pl.BlockSpec((BQ, 2), lambda i, h: (i, 0)),
            pl.BlockSpec((2, S), lambda i, h: (0, 0)),
        ],
        out_specs=pl.BlockSpec((1, BQ, D), lambda i, h: (h, i, 0)),
        out_shape=jax.ShapeDtypeStruct((H, S, D), jnp.float32),
        scratch_shapes=[pltpu.VMEM((BQ, S), jnp.float32)],
    )(q3, k3, v3, span_col, span_row, caus_col, coords, coords_t)
    return out[None]
